# Initial kernel scaffold; baseline (speedup 1.0000x reference)
import jax
import jax.numpy as jnp
from jax.experimental import pallas as pl


def kernel(x, table):
    raise NotImplementedError("write your pallas kernel here")


# speedup vs baseline: 1.0013x; 1.0013x over previous
import functools
import math

import jax
import jax.numpy as jnp
from jax import lax
from jax.experimental import pallas as pl
from jax.experimental.pallas import tpu as pltpu
from jax.experimental.pallas import tpu_sc as plsc

NC = 2
NS = 16
LANES = 16

CHUNK = 32


@functools.partial(jax.jit, static_argnames=("n_rows", "d"))
def _emb_lookup(idx, table, n_rows, d):
    nw = NC * NS
    per_w = n_rows // nw
    n_chunks = per_w // CHUNK
    scale = float(math.sqrt(d))
    mesh = plsc.VectorSubcoreMesh(
        core_axis_name="c", subcore_axis_name="s",
        num_cores=NC, num_subcores=NS)

    @functools.partial(
        pl.kernel,
        out_type=jax.ShapeDtypeStruct((n_rows, d), jnp.float32),
        mesh=mesh,
        scratch_types=[
            pltpu.VMEM((per_w,), jnp.int32),
            pltpu.VMEM((CHUNK, d), jnp.float32),
            pltpu.SemaphoreType.DMA,
        ],
    )
    def k(idx_hbm, table_hbm, out_hbm, idx_v, buf, sem):
        wid = lax.axis_index("s") * NC + lax.axis_index("c")
        base = wid * per_w
        pltpu.sync_copy(idx_hbm.at[pl.ds(base, per_w)], idx_v)
        for g in range(n_chunks):
            pltpu.async_copy(
                table_hbm.at[idx_v.at[pl.ds(g * CHUNK, CHUNK)]], buf, sem
            ).wait()

            def scale_row(r, carry):
                for j in range(d // LANES):
                    sl = pl.ds(j * LANES, LANES)
                    buf[r, sl] = buf[r, sl] * scale
                return carry

            lax.fori_loop(0, CHUNK, scale_row, 0)
            pltpu.sync_copy(buf, out_hbm.at[pl.ds(base + g * CHUNK, CHUNK)])

    return k(idx, table)


def kernel(x, table):
    b, s = x.shape
    v, d = table.shape
    idx = x.reshape(-1).astype(jnp.int32)
    out = _emb_lookup(idx, table, n_rows=b * s, d=d)
    return out.reshape(b, s, d)


# speedup vs baseline: 1.2901x; 1.2884x over previous
import functools
import math

import jax
import jax.numpy as jnp
from jax import lax
from jax.experimental import pallas as pl
from jax.experimental.pallas import tpu as pltpu
from jax.experimental.pallas import tpu_sc as plsc

NC = 2
NS = 16
LANES = 16

CHUNK = 32
NBUF = 3


@functools.partial(jax.jit, static_argnames=("n_rows", "d"))
def _emb_lookup(idx, table, n_rows, d):
    nw = NC * NS
    per_w = n_rows // nw
    n_chunks = per_w // CHUNK
    nbuf = min(NBUF, n_chunks)
    scale = float(math.sqrt(d))
    mesh = plsc.VectorSubcoreMesh(
        core_axis_name="c", subcore_axis_name="s",
        num_cores=NC, num_subcores=NS)

    @functools.partial(
        pl.kernel,
        out_type=jax.ShapeDtypeStruct((n_rows, d), jnp.float32),
        mesh=mesh,
        scratch_types=(
            [pltpu.VMEM((per_w,), jnp.int32)]
            + [pltpu.VMEM((CHUNK, d), jnp.float32) for _ in range(nbuf)]
            + [pltpu.SemaphoreType.DMA for _ in range(2 * nbuf)]
        ),
    )
    def k(idx_hbm, table_hbm, out_hbm, idx_v, *rest):
        bufs = rest[:nbuf]
        gsems = rest[nbuf:2 * nbuf]
        wsems = rest[2 * nbuf:]
        wid = lax.axis_index("s") * NC + lax.axis_index("c")
        base = wid * per_w
        pltpu.sync_copy(idx_hbm.at[pl.ds(base, per_w)], idx_v)

        def issue_gather(g, i):
            return pltpu.async_copy(
                table_hbm.at[idx_v.at[pl.ds(g * CHUNK, CHUNK)]],
                bufs[i], gsems[i])

        def scale_buf(buf):
            def scale_row(r, carry):
                for j in range(d // LANES):
                    sl = pl.ds(j * LANES, LANES)
                    buf[r, sl] = buf[r, sl] * scale
                return carry
            lax.fori_loop(0, CHUNK, scale_row, 0)

        ghandles = {}
        whandles = {}
        for g in range(nbuf):
            ghandles[g] = issue_gather(g, g)
        for g in range(n_chunks):
            i = g % nbuf
            ghandles[g].wait()
            scale_buf(bufs[i])
            whandles[g] = pltpu.async_copy(
                bufs[i], out_hbm.at[pl.ds(base + g * CHUNK, CHUNK)], wsems[i])
            nxt = g + nbuf
            if nxt < n_chunks:
                whandles[g].wait()
                ghandles[nxt] = issue_gather(nxt, i)
        for g in range(max(0, n_chunks - nbuf), n_chunks):
            whandles[g].wait()

    return k(idx, table)


def kernel(x, table):
    b, s = x.shape
    v, d = table.shape
    idx = x.reshape(-1).astype(jnp.int32)
    out = _emb_lookup(idx, table, n_rows=b * s, d=d)
    return out.reshape(b, s, d)


# speedup vs baseline: 1.3653x; 1.0583x over previous
import functools
import math

import jax
import jax.numpy as jnp
from jax import lax
from jax.experimental import pallas as pl
from jax.experimental.pallas import tpu as pltpu
from jax.experimental.pallas import tpu_sc as plsc

NC = 2
NS = 16
LANES = 16

CHUNK = 32
NBUF = 3


@functools.partial(jax.jit, static_argnames=("n_rows", "d"))
def _emb_lookup(idx, table, n_rows, d):
    nw = NC * NS
    per_w = n_rows // nw
    n_chunks = per_w // CHUNK
    nbuf = min(NBUF, n_chunks)
    scale = float(math.sqrt(d))
    mesh = plsc.VectorSubcoreMesh(
        core_axis_name="c", subcore_axis_name="s",
        num_cores=NC, num_subcores=NS)

    @functools.partial(
        pl.kernel,
        out_type=jax.ShapeDtypeStruct((n_rows, d), jnp.float32),
        mesh=mesh,
        scratch_types=(
            [pltpu.VMEM((per_w,), jnp.int32)]
            + [pltpu.VMEM((CHUNK, d), jnp.float32) for _ in range(nbuf)]
            + [pltpu.SemaphoreType.DMA for _ in range(2 * nbuf)]
        ),
    )
    def k(idx_hbm, table_hbm, out_hbm, idx_v, *rest):
        bufs = rest[:nbuf]
        gsems = rest[nbuf:2 * nbuf]
        wsems = rest[2 * nbuf:]
        wid = lax.axis_index("s") * NC + lax.axis_index("c")
        base = wid * per_w
        pltpu.sync_copy(idx_hbm.at[pl.ds(base, per_w)], idx_v)

        def issue_gather(g, i):
            return pltpu.async_copy(
                table_hbm.at[idx_v.at[pl.ds(g * CHUNK, CHUNK)]],
                bufs[i], gsems[i])

        def scale_buf(buf):
            def scale_row(r, carry):
                for j in range(d // LANES):
                    sl = pl.ds(j * LANES, LANES)
                    buf[r, sl] = buf[r, sl] * scale
                return carry
            lax.fori_loop(0, CHUNK, scale_row, 0)

        ghandles = {}
        whandles = {}
        for g in range(nbuf):
            ghandles[g] = issue_gather(g, g)
        for g in range(n_chunks):
            i = g % nbuf
            ghandles[g].wait()
            scale_buf(bufs[i])
            whandles[g] = pltpu.async_copy(
                bufs[i], out_hbm.at[pl.ds(base + g * CHUNK, CHUNK)], wsems[i])
            prev = g - 1
            if prev >= 0 and prev + nbuf < n_chunks:
                whandles[prev].wait()
                ghandles[prev + nbuf] = issue_gather(prev + nbuf, prev % nbuf)
        for g in range(max(0, n_chunks - nbuf), n_chunks):
            whandles[g].wait()

    return k(idx, table)


def kernel(x, table):
    b, s = x.shape
    v, d = table.shape
    idx = x.reshape(-1).astype(jnp.int32)
    out = _emb_lookup(idx, table, n_rows=b * s, d=d)
    return out.reshape(b, s, d)


# speedup vs baseline: 1.3740x; 1.0064x over previous
import functools
import math

import jax
import jax.numpy as jnp
from jax import lax
from jax.experimental import pallas as pl
from jax.experimental.pallas import tpu as pltpu
from jax.experimental.pallas import tpu_sc as plsc

NC = 2
NS = 16
LANES = 16

CHUNK = 16
NBUF = 6
WDEFER = 2


@functools.partial(jax.jit, static_argnames=("n_rows", "d"))
def _emb_lookup(idx, table, n_rows, d):
    nw = NC * NS
    per_w = n_rows // nw
    n_chunks = per_w // CHUNK
    nbuf = min(NBUF, n_chunks)
    scale = float(math.sqrt(d))
    mesh = plsc.VectorSubcoreMesh(
        core_axis_name="c", subcore_axis_name="s",
        num_cores=NC, num_subcores=NS)

    @functools.partial(
        pl.kernel,
        out_type=jax.ShapeDtypeStruct((n_rows, d), jnp.float32),
        mesh=mesh,
        scratch_types=(
            [pltpu.VMEM((per_w,), jnp.int32)]
            + [pltpu.VMEM((CHUNK, d), jnp.float32) for _ in range(nbuf)]
            + [pltpu.SemaphoreType.DMA for _ in range(2 * nbuf)]
        ),
    )
    def k(idx_hbm, table_hbm, out_hbm, idx_v, *rest):
        bufs = rest[:nbuf]
        gsems = rest[nbuf:2 * nbuf]
        wsems = rest[2 * nbuf:]
        wid = lax.axis_index("s") * NC + lax.axis_index("c")
        base = wid * per_w
        pltpu.sync_copy(idx_hbm.at[pl.ds(base, per_w)], idx_v)

        def issue_gather(g, i):
            return pltpu.async_copy(
                table_hbm.at[idx_v.at[pl.ds(g * CHUNK, CHUNK)]],
                bufs[i], gsems[i])

        def scale_buf(buf):
            def scale_row(r, carry):
                for j in range(d // LANES):
                    sl = pl.ds(j * LANES, LANES)
                    buf[r, sl] = buf[r, sl] * scale
                return carry
            lax.fori_loop(0, CHUNK, scale_row, 0)

        ghandles = {}
        whandles = {}
        for g in range(nbuf):
            ghandles[g] = issue_gather(g, g)
        for g in range(n_chunks):
            i = g % nbuf
            ghandles[g].wait()
            scale_buf(bufs[i])
            whandles[g] = pltpu.async_copy(
                bufs[i], out_hbm.at[pl.ds(base + g * CHUNK, CHUNK)], wsems[i])
            prev = g - WDEFER
            if prev >= 0 and prev + nbuf < n_chunks:
                whandles[prev].wait()
                ghandles[prev + nbuf] = issue_gather(prev + nbuf, prev % nbuf)
        for g in range(max(0, n_chunks - nbuf), n_chunks):
            whandles[g].wait()

    return k(idx, table)


def kernel(x, table):
    b, s = x.shape
    v, d = table.shape
    idx = x.reshape(-1).astype(jnp.int32)
    out = _emb_lookup(idx, table, n_rows=b * s, d=d)
    return out.reshape(b, s, d)
